# baseline (speedup 1.0000x reference)
import functools

import jax
import jax.numpy as jnp
from jax import lax
from jax.experimental import pallas as pl
from jax.experimental.pallas import tpu as pltpu
from jax.experimental.pallas import tpu_sc as plsc

B, H, N_SMALL, N, D = 8, 4, 1024, 2048, 128
SUB = 128
NSUB = N_SMALL // SUB


def _copy_body(src_ref, dst_ref):
    dst_ref[...] = src_ref[...]


def _tc_copy(x2d, block_rows):
    rows, cols = x2d.shape
    return pl.pallas_call(
        _copy_body,
        grid=(rows // block_rows,),
        in_specs=[pl.BlockSpec((block_rows, cols), lambda i: (i, 0))],
        out_specs=pl.BlockSpec((block_rows, cols), lambda i: (i, 0)),
        out_shape=jax.ShapeDtypeStruct((rows, cols), x2d.dtype),
    )(x2d)


def _sc_scatter(h, pre_h, idx3, out_ref):
    mesh = plsc.VectorSubcoreMesh(core_axis_name="c", subcore_axis_name="s")

    @functools.partial(
        pl.kernel,
        mesh=mesh,
        out_type=(),
        scratch_types=[
            pltpu.VMEM((SUB, D), jnp.float32),
            pltpu.VMEM((SUB, D), jnp.float32),
            pltpu.VMEM((NSUB, SUB), jnp.int32),
            pltpu.SemaphoreType.DMA,
        ],
    )
    def k(h_hbm, pre_hbm, idx_hbm, out_hbm, pbuf, hbuf, idxv, sem):
        cid = lax.axis_index("c")
        sid = lax.axis_index("s")
        wid = sid * 2 + cid
        b = wid // H
        hh = wid % H

        pltpu.sync_copy(idx_hbm.at[b], idxv)
        for s in range(NSUB):
            pltpu.async_copy(
                pre_hbm.at[b, hh].at[idxv.at[s]], pbuf, sem,
            ).wait()
            pltpu.sync_copy(h_hbm.at[b, hh, pl.ds(s * SUB, SUB)], hbuf)

            def addrow(r, _):
                for l in range(D // 16):
                    sl = pl.ds(l * 16, 16)
                    plsc.addupdate(pbuf.at[r, sl], hbuf[r, sl])
                return _
            lax.fori_loop(0, SUB, addrow, 0, unroll=False)

            pltpu.async_copy(
                pbuf, out_hbm.at[b, hh].at[idxv.at[s]], sem,
            ).wait()

    k(h, pre_h, idx3, out_ref)


def kernel(g, h, pre_h, idx):
    idx3 = idx.astype(jnp.int32).reshape(B, NSUB, SUB)
    out0 = _tc_copy(pre_h.reshape(2048, 4096), 256)
    out0 = out0.reshape(B, H, N, D)
    out_ref = jax.new_ref(out0)
    _sc_scatter(h, pre_h, idx3, out_ref)
    return (g, out_ref[...])


# speedup vs baseline: 1.5215x; 1.5215x over previous
import functools

import jax
import jax.numpy as jnp
from jax import lax
from jax.experimental import pallas as pl
from jax.experimental.pallas import tpu as pltpu
from jax.experimental.pallas import tpu_sc as plsc

B, H, N_SMALL, N, D = 8, 4, 1024, 2048, 128
CA = 256
SUB = 128
NSUB = N_SMALL // SUB


def _sc_unpool(h, pre_h, idx3):
    mesh = plsc.VectorSubcoreMesh(core_axis_name="c", subcore_axis_name="s")

    @functools.partial(
        pl.kernel,
        mesh=mesh,
        out_type=jax.ShapeDtypeStruct((B, H, N, D), jnp.float32),
        scratch_types=[
            pltpu.VMEM((CA, D), jnp.float32),
            pltpu.VMEM((CA, D), jnp.float32),
            pltpu.VMEM((NSUB, SUB), jnp.int32),
        ] + [pltpu.SemaphoreType.DMA] * 8,
    )
    def k(h_hbm, pre_hbm, idx_hbm, out_hbm, buf0, buf1, idxv,
          sa0, sa1, sg0, sg1, sh0, sh1, ss0, ss1):
        cid = lax.axis_index("c")
        sid = lax.axis_index("s")
        wid = sid * 2 + cid
        b = wid // H
        hh = wid % H

        pltpu.sync_copy(idx_hbm.at[b], idxv)

        bufs = (buf0, buf1)
        asems = (sa0, sa1)
        a_cps = [None, None]
        for c in range(N // CA):
            bi = c & 1
            if a_cps[bi] is not None:
                a_cps[bi].wait()
            pltpu.sync_copy(pre_hbm.at[b, hh, pl.ds(c * CA, CA)], bufs[bi])
            a_cps[bi] = pltpu.async_copy(
                bufs[bi], out_hbm.at[b, hh, pl.ds(c * CA, CA)], asems[bi])
        a_cps[0].wait()
        a_cps[1].wait()

        pviews = (buf0.at[pl.ds(0, SUB)], buf1.at[pl.ds(0, SUB)])
        hviews = (buf0.at[pl.ds(SUB, SUB)], buf1.at[pl.ds(SUB, SUB)])
        gsems = (sg0, sg1)
        hsems = (sh0, sh1)
        ssems = (ss0, ss1)
        g_cps = [None, None]
        h_cps = [None, None]
        s_cps = [None, None]

        def start(s):
            bi = s & 1
            g_cps[bi] = pltpu.async_copy(
                pre_hbm.at[b, hh].at[idxv.at[s]], pviews[bi], gsems[bi])
            h_cps[bi] = pltpu.async_copy(
                h_hbm.at[b, hh, pl.ds(s * SUB, SUB)], hviews[bi], hsems[bi])

        start(0)
        for s in range(NSUB):
            bi = s & 1
            g_cps[bi].wait()
            h_cps[bi].wait()

            def addrow(r, _):
                for l in range(D // 16):
                    sl = pl.ds(l * 16, 16)
                    plsc.addupdate(pviews[bi].at[r, sl], hviews[bi][r, sl])
                return _
            lax.fori_loop(0, SUB, addrow, 0, unroll=False)

            if s + 1 < NSUB:
                ob = (s + 1) & 1
                if s_cps[ob] is not None:
                    s_cps[ob].wait()
                start(s + 1)
            s_cps[bi] = pltpu.async_copy(
                pviews[bi], out_hbm.at[b, hh].at[idxv.at[s]], ssems[bi])
        s_cps[0].wait()
        s_cps[1].wait()

    return k(h, pre_h, idx3)


def kernel(g, h, pre_h, idx):
    idx3 = idx.astype(jnp.int32).reshape(B, NSUB, SUB)
    new_h = _sc_unpool(h, pre_h, idx3)
    return (g, new_h)


# speedup vs baseline: 1.7073x; 1.1221x over previous
import functools

import jax
import jax.numpy as jnp
from jax import lax
from jax.experimental import pallas as pl
from jax.experimental.pallas import tpu as pltpu
from jax.experimental.pallas import tpu_sc as plsc

B, H, N_SMALL, N, D = 8, 4, 1024, 2048, 128
CA = 256
SUB = 128
NSUB = N_SMALL // SUB


def _copy_body(src_ref, dst_ref):
    dst_ref[...] = src_ref[...]


def _tc_copy(x2d, block_rows):
    rows, cols = x2d.shape
    return pl.pallas_call(
        _copy_body,
        grid=(rows // block_rows,),
        in_specs=[pl.BlockSpec((block_rows, cols), lambda i: (i, 0))],
        out_specs=pl.BlockSpec((block_rows, cols), lambda i: (i, 0)),
        out_shape=jax.ShapeDtypeStruct((rows, cols), x2d.dtype),
    )(x2d)


def _sc_unpool(h, pre_h, idx3):
    mesh = plsc.VectorSubcoreMesh(core_axis_name="c", subcore_axis_name="s")

    @functools.partial(
        pl.kernel,
        mesh=mesh,
        out_type=jax.ShapeDtypeStruct((B, H, N, D), jnp.float32),
        scratch_types=[
            pltpu.VMEM((CA, D), jnp.float32),
            pltpu.VMEM((CA, D), jnp.float32),
            pltpu.VMEM((NSUB, SUB), jnp.int32),
        ] + [pltpu.SemaphoreType.DMA] * 8,
    )
    def k(h_hbm, pre_hbm, idx_hbm, out_hbm, buf0, buf1, idxv,
          sa0, sa1, sg0, sg1, sh0, sh1, ss0, ss1):
        cid = lax.axis_index("c")
        sid = lax.axis_index("s")
        wid = sid * 2 + cid
        b = wid // H
        hh = wid % H

        pltpu.sync_copy(idx_hbm.at[b], idxv)

        bufs = (buf0, buf1)
        asems = (sa0, sa1)
        a_cps = [None, None]
        for c in range(N // CA):
            bi = c & 1
            if a_cps[bi] is not None:
                a_cps[bi].wait()
            pltpu.sync_copy(pre_hbm.at[b, hh, pl.ds(c * CA, CA)], bufs[bi])
            a_cps[bi] = pltpu.async_copy(
                bufs[bi], out_hbm.at[b, hh, pl.ds(c * CA, CA)], asems[bi])
        a_cps[0].wait()
        a_cps[1].wait()

        pviews = (buf0.at[pl.ds(0, SUB)], buf1.at[pl.ds(0, SUB)])
        hviews = (buf0.at[pl.ds(SUB, SUB)], buf1.at[pl.ds(SUB, SUB)])
        gsems = (sg0, sg1)
        hsems = (sh0, sh1)
        ssems = (ss0, ss1)
        g_cps = [None, None]
        h_cps = [None, None]
        s_cps = [None, None]

        def start(s):
            bi = s & 1
            g_cps[bi] = pltpu.async_copy(
                pre_hbm.at[b, hh].at[idxv.at[s]], pviews[bi], gsems[bi])
            h_cps[bi] = pltpu.async_copy(
                h_hbm.at[b, hh, pl.ds(s * SUB, SUB)], hviews[bi], hsems[bi])

        start(0)
        for s in range(NSUB):
            bi = s & 1
            g_cps[bi].wait()
            h_cps[bi].wait()

            def addrow(r, _):
                for l in range(D // 16):
                    sl = pl.ds(l * 16, 16)
                    plsc.addupdate(pviews[bi].at[r, sl], hviews[bi][r, sl])
                return _
            lax.fori_loop(0, SUB, addrow, 0, unroll=False)

            if s + 1 < NSUB:
                ob = (s + 1) & 1
                if s_cps[ob] is not None:
                    s_cps[ob].wait()
                start(s + 1)
            s_cps[bi] = pltpu.async_copy(
                pviews[bi], out_hbm.at[b, hh].at[idxv.at[s]], ssems[bi])
        s_cps[0].wait()
        s_cps[1].wait()

    return k(h, pre_h, idx3)


def kernel(g, h, pre_h, idx):
    idx3 = idx.astype(jnp.int32).reshape(B, NSUB, SUB)
    new_h = _sc_unpool(h, pre_h, idx3)
    g_out = _tc_copy(g.reshape(B * N, N), 1024).reshape(B, N, N)
    return (g_out, new_h)


# speedup vs baseline: 1.7585x; 1.0300x over previous
import functools

import jax
import jax.numpy as jnp
from jax import lax
from jax.experimental import pallas as pl
from jax.experimental.pallas import tpu as pltpu
from jax.experimental.pallas import tpu_sc as plsc

B, H, N_SMALL, N, D = 8, 4, 1024, 2048, 128
CA = 240
W = 256
CHUNKS = [(i * CA, CA) for i in range(8)] + [(8 * CA, N - 8 * CA)]
THRESH = [n0 for n0, _ in CHUNKS[1:]]
LANES = 16


def _copy_body(src_ref, dst_ref):
    dst_ref[...] = src_ref[...]


def _tc_copy(x2d, block_rows):
    rows, cols = x2d.shape
    return pl.pallas_call(
        _copy_body,
        grid=(rows // block_rows,),
        in_specs=[pl.BlockSpec((block_rows, cols), lambda i: (i, 0))],
        out_specs=pl.BlockSpec((block_rows, cols), lambda i: (i, 0)),
        out_shape=jax.ShapeDtypeStruct((rows, cols), x2d.dtype),
    )(x2d)


def _sc_unpool(h, pre_h, idx32):
    mesh = plsc.VectorSubcoreMesh(core_axis_name="c", subcore_axis_name="s")

    @functools.partial(
        pl.kernel,
        mesh=mesh,
        out_type=jax.ShapeDtypeStruct((B, H, N, D), jnp.float32),
        scratch_types=[
            pltpu.VMEM((CA, D), jnp.float32),
            pltpu.VMEM((CA, D), jnp.float32),
            pltpu.VMEM((W, D), jnp.float32),
            pltpu.VMEM((W, D), jnp.float32),
            pltpu.VMEM((N_SMALL + LANES,), jnp.int32),
        ] + [pltpu.SemaphoreType.DMA] * 6,
    )
    def k(h_hbm, pre_hbm, idx_hbm, out_hbm, ob0, ob1, hb0, hb1, idxv,
          sp0, sp1, sh0, sh1, sw0, sw1):
        cid = lax.axis_index("c")
        sid = lax.axis_index("s")
        wid = sid * 2 + cid
        b = wid // H
        hh = wid % H

        pltpu.sync_copy(idx_hbm.at[b], idxv.at[pl.ds(0, N_SMALL)])

        def searchsorted(t):
            def bs_body(step, lo):
                probe = lo + jnp.int32(2 ** (9 - step))
                v = idxv[pl.ds(probe - 1, LANES)][0]
                return jnp.where((probe <= N_SMALL) & (v < t), probe, lo)
            return lax.fori_loop(0, 10, bs_body, jnp.int32(0), unroll=True)
        js = [jnp.int32(0)] + [searchsorted(jnp.int32(t)) for t in THRESH]
        j0a = [(jnp.minimum(j, jnp.int32(N_SMALL - W)) // 8) * 8
               for j in js]

        obufs = (ob0, ob1)
        hbufs = (hb0, hb1)
        psems = (sp0, sp1)
        hsems = (sh0, sh1)
        wsems = (sw0, sw1)
        p_cps = [None, None]
        h_cps = [None, None]
        w_cps = [None, None]

        def start_reads(c):
            bi = c & 1
            n0, cc = CHUNKS[c]
            p_cps[bi] = pltpu.async_copy(
                pre_hbm.at[b, hh, pl.ds(n0, cc)],
                obufs[bi].at[pl.ds(0, cc)], psems[bi])
            h_cps[bi] = pltpu.async_copy(
                h_hbm.at[b, hh, pl.ds(j0a[c], W)], hbufs[bi], hsems[bi])

        start_reads(0)
        for c in range(len(CHUNKS)):
            bi = c & 1
            n0, cc = CHUNKS[c]
            p_cps[bi].wait()
            h_cps[bi].wait()
            if c + 1 < len(CHUNKS):
                ob = (c + 1) & 1
                if w_cps[ob] is not None:
                    w_cps[ob].wait()
                start_reads(c + 1)

            obuf = obufs[bi]
            hbuf = hbufs[bi]
            base = j0a[c]

            def place(kk, carry):
                tv = idxv[pl.ds(base + kk * LANES, LANES)] - n0
                for r in range(LANES):
                    t_r = tv[r]
                    @pl.when((t_r >= 0) & (t_r < cc))
                    def _do(t_r=t_r, kk=kk, r=r):
                        for l in range(D // LANES):
                            sl = pl.ds(l * LANES, LANES)
                            plsc.addupdate(obuf.at[t_r, sl],
                                           hbuf[kk * LANES + r, sl])
                return carry
            lax.fori_loop(0, W // LANES, place, 0, unroll=False)

            w_cps[bi] = pltpu.async_copy(
                obuf.at[pl.ds(0, cc)],
                out_hbm.at[b, hh, pl.ds(n0, cc)], wsems[bi])
        w_cps[0].wait()
        w_cps[1].wait()

    return k(h, pre_h, idx32)


def kernel(g, h, pre_h, idx):
    idx32 = idx.astype(jnp.int32)
    new_h = _sc_unpool(h, pre_h, idx32)
    g_out = _tc_copy(g.reshape(B * N, N), 1024).reshape(B, N, N)
    return (g_out, new_h)


# speedup vs baseline: 1.7644x; 1.0033x over previous
import functools

import jax
import jax.numpy as jnp
from jax import lax
from jax.experimental import pallas as pl
from jax.experimental.pallas import tpu as pltpu
from jax.experimental.pallas import tpu_sc as plsc

B, H, N_SMALL, N, D = 8, 4, 1024, 2048, 128
CA = 240
W = 256
CHUNKS = [(i * CA, CA) for i in range(8)] + [(8 * CA, N - 8 * CA)]
THRESH = [n0 for n0, _ in CHUNKS[1:]]
LANES = 16


def _copy_body(src_ref, dst_ref):
    dst_ref[...] = src_ref[...]


def _tc_copy(x2d, block_rows):
    rows, cols = x2d.shape
    return pl.pallas_call(
        _copy_body,
        grid=(rows // block_rows,),
        in_specs=[pl.BlockSpec((block_rows, cols), lambda i: (i, 0))],
        out_specs=pl.BlockSpec((block_rows, cols), lambda i: (i, 0)),
        out_shape=jax.ShapeDtypeStruct((rows, cols), x2d.dtype),
    )(x2d)


def _sc_unpool(h, pre_h, idx32):
    mesh = plsc.VectorSubcoreMesh(core_axis_name="c", subcore_axis_name="s")

    @functools.partial(
        pl.kernel,
        mesh=mesh,
        out_type=jax.ShapeDtypeStruct((B, H, N, D), jnp.float32),
        scratch_types=[
            pltpu.VMEM((CA, D), jnp.float32),
            pltpu.VMEM((CA, D), jnp.float32),
            pltpu.VMEM((W, D), jnp.float32),
            pltpu.VMEM((W, D), jnp.float32),
            pltpu.VMEM((N_SMALL + LANES,), jnp.int32),
        ] + [pltpu.SemaphoreType.DMA] * 6,
    )
    def k(h_hbm, pre_hbm, idx_hbm, out_hbm, ob0, ob1, hb0, hb1, idxv,
          sp0, sp1, sh0, sh1, sw0, sw1):
        cid = lax.axis_index("c")
        sid = lax.axis_index("s")
        wid = sid * 2 + cid
        b = wid // H
        hh = wid % H

        pltpu.sync_copy(idx_hbm.at[b], idxv.at[pl.ds(0, N_SMALL)])

        def searchsorted(t):
            def bs_body(step, lo):
                probe = lo + jnp.int32(2 ** (9 - step))
                v = idxv[pl.ds(probe - 1, LANES)][0]
                return jnp.where((probe <= N_SMALL) & (v < t), probe, lo)
            return lax.fori_loop(0, 10, bs_body, jnp.int32(0), unroll=True)
        js = [jnp.int32(0)] + [searchsorted(jnp.int32(t)) for t in THRESH]
        j0a = [(jnp.minimum(j, jnp.int32(N_SMALL - W)) // 8) * 8
               for j in js]

        obufs = (ob0, ob1)
        hbufs = (hb0, hb1)
        psems = (sp0, sp1)
        hsems = (sh0, sh1)
        wsems = (sw0, sw1)
        p_cps = [None, None]
        h_cps = [None, None]
        w_cps = [None, None]

        def start_reads(c):
            bi = c & 1
            n0, cc = CHUNKS[c]
            p_cps[bi] = pltpu.async_copy(
                pre_hbm.at[b, hh, pl.ds(n0, cc)],
                obufs[bi].at[pl.ds(0, cc)], psems[bi])
            h_cps[bi] = pltpu.async_copy(
                h_hbm.at[b, hh, pl.ds(j0a[c], W)], hbufs[bi], hsems[bi])

        start_reads(0)
        for c in range(len(CHUNKS)):
            bi = c & 1
            n0, cc = CHUNKS[c]
            p_cps[bi].wait()
            h_cps[bi].wait()
            if c + 1 < len(CHUNKS):
                ob = (c + 1) & 1
                if w_cps[ob] is not None:
                    w_cps[ob].wait()
                start_reads(c + 1)

            obuf = obufs[bi]
            hbuf = hbufs[bi]
            base = j0a[c]
            je = js[c + 1] if c + 1 < len(CHUNKS) else jnp.int32(N_SMALL)

            def place(j, carry):
                t = idxv[pl.ds(j, LANES)][0] - n0
                hr = j - base
                for l in range(D // LANES):
                    sl = pl.ds(l * LANES, LANES)
                    plsc.addupdate(obuf.at[t, sl], hbuf[hr, sl])
                return carry
            lax.fori_loop(js[c], je, place, 0, unroll=False)

            w_cps[bi] = pltpu.async_copy(
                obuf.at[pl.ds(0, cc)],
                out_hbm.at[b, hh, pl.ds(n0, cc)], wsems[bi])
        w_cps[0].wait()
        w_cps[1].wait()

    return k(h, pre_h, idx32)


def kernel(g, h, pre_h, idx):
    idx32 = idx.astype(jnp.int32)
    new_h = _sc_unpool(h, pre_h, idx32)
    g_out = _tc_copy(g.reshape(B * N, N), 1024).reshape(B, N, N)
    return (g_out, new_h)
